# baseline (speedup 1.0000x reference)
import functools

import jax
import jax.numpy as jnp
from jax import lax
from jax.experimental import pallas as pl
from jax.experimental.pallas import tpu as pltpu
from jax.experimental.pallas import tpu_sc as plsc

_T = 8192
_D = 2048
_E = 64
_TOP_K = 8
_N_GROUPS = 8
_EPG = _E // _N_GROUPS
_TOPK_GROUPS = 4
_SCALE = 2.5

_BLK_T = 1024
_NEG = -1e30

_NCHUNK = 4
_CT = _T // _NCHUNK

_NC = 2
_NS = 16
_NW = _NC * _NS
_TPW = _CT // _NW
_PAIRS = _TPW // 2


def _score_block(x_ref, w_ref, o_ref):
    o_ref[...] = jax.nn.sigmoid(
        jnp.dot(x_ref[...], w_ref[...], preferred_element_type=jnp.float32))


def _shuf(v, perm):
    return lax.gather(
        v, perm[:, None],
        dimension_numbers=lax.GatherDimensionNumbers(
            offset_dims=(), collapsed_slice_dims=(0,), start_index_map=(0,)),
        slice_sizes=(1,),
        mode=lax.GatherScatterMode.PROMISE_IN_BOUNDS)


def _route_sc_body(scores_hbm, bias_hbm, wout_hbm, iout_hbm,
                   sc_v, bias_v, w_v, i_v):
    wid = lax.axis_index("s") * _NC + lax.axis_index("c")
    base = wid * _TPW
    pltpu.sync_copy(scores_hbm.at[pl.ds(base, _TPW), :], sc_v)
    pltpu.sync_copy(bias_hbm, bias_v)

    lane = lax.iota(jnp.int32, 16)
    bias_regs = [bias_v[pl.ds(i * 16, 16)] for i in range(4)]
    neg = jnp.full((16,), _NEG, jnp.float32)

    def token_route(t):
        ss = [sc_v[t, pl.ds(i * 16, 16)] for i in range(4)]
        bs = [s + b for s, b in zip(ss, bias_regs)]

        gsums = []
        for i in range(4):
            m1 = bs[i]
            m2 = neg
            for d in (1, 2, 4):
                perm = lane ^ d
                pm1 = _shuf(m1, perm)
                pm2 = _shuf(m2, perm)
                lo = jnp.minimum(m1, pm1)
                m1 = jnp.maximum(m1, pm1)
                m2 = jnp.maximum(lo, jnp.maximum(m2, pm2))
            gsums.append(m1 + m2)

        gs = neg
        for i in range(4):
            bi = _shuf(gsums[i], (lane & 1) * 8)
            gs = jnp.where((lane >> 1) == i, bi, gs)

        _, gsv = plsc.sort_key_val(gs, lane, descending=True)
        sel = [_shuf(gsv, jnp.full((16,), k, jnp.int32)) for k in range(4)]

        sorted_kv = []
        for i in range(4):
            gid = 2 * i + (lane >> 3)
            keep = ((gid == sel[0]) | (gid == sel[1])
                    | (gid == sel[2]) | (gid == sel[3]))
            masked = jnp.where(keep, bs[i], 0.0)
            sorted_kv.append(
                plsc.sort_key_val(masked, lane + 16 * i, descending=True))

        def merge(a, b):
            ka, va = a
            kb, vb = b
            kab = jnp.where(lane < 8, ka, _shuf(kb, lane & 7))
            vab = jnp.where(lane < 8, va, _shuf(vb, lane & 7))
            return plsc.sort_key_val(kab, vab, descending=True)

        _, mv = merge(merge(sorted_kv[0], sorted_kv[1]),
                      merge(sorted_kv[2], sorted_kv[3]))

        tvec = jnp.full((16,), t, jnp.int32)
        wv = plsc.load_gather(sc_v, [tvec, mv])
        wm = jnp.where(lane < 8, wv, 0.0)
        tot = lax.broadcast(jnp.sum(wm), (16,))
        wout = wm * (_SCALE / (tot + 1e-20))
        return wout, mv

    def pair_body(j, carry):
        w0, i0 = token_route(2 * j)
        w1, i1 = token_route(2 * j + 1)
        pw = jnp.where(lane < 8, w0, _shuf(w1, lane & 7))
        pi = jnp.where(lane < 8, i0, _shuf(i1, lane & 7))
        w_v[pl.ds(j * 16, 16)] = pw
        i_v[pl.ds(j * 16, 16)] = pi
        return carry

    lax.fori_loop(0, _PAIRS, pair_body, 0)

    pltpu.sync_copy(w_v, wout_hbm.at[pl.ds(base * _TOP_K, _TPW * _TOP_K)])
    pltpu.sync_copy(i_v, iout_hbm.at[pl.ds(base * _TOP_K, _TPW * _TOP_K)])


_route_sc = functools.partial(
    pl.kernel,
    mesh=plsc.VectorSubcoreMesh(core_axis_name="c", subcore_axis_name="s"),
    compiler_params=pltpu.CompilerParams(needs_layout_passes=False),
    out_type=[
        jax.ShapeDtypeStruct((_CT * _TOP_K,), jnp.float32),
        jax.ShapeDtypeStruct((_CT * _TOP_K,), jnp.int32),
    ],
    scratch_types=[
        pltpu.VMEM((_TPW, _E), jnp.float32),
        pltpu.VMEM((_E,), jnp.float32),
        pltpu.VMEM((_TPW * _TOP_K,), jnp.float32),
        pltpu.VMEM((_TPW * _TOP_K,), jnp.int32),
    ],
)(_route_sc_body)


@jax.jit
def kernel(x, kernel_DE, bias_E):
    x = jnp.asarray(x, jnp.float32)
    w_parts = []
    i_parts = []
    for c in range(_NCHUNK):
        scores_c = pl.pallas_call(
            _score_block,
            grid=(_CT // _BLK_T,),
            in_specs=[
                pl.BlockSpec((_BLK_T, _D),
                             functools.partial(lambda c, i: (c * (_CT // _BLK_T) + i, 0), c)),
                pl.BlockSpec((_D, _E), lambda i: (0, 0)),
            ],
            out_specs=pl.BlockSpec((_BLK_T, _E), lambda i: (i, 0)),
            out_shape=jax.ShapeDtypeStruct((_CT, _E), jnp.float32),
        )(x, kernel_DE)
        wf, jf = _route_sc(scores_c, bias_E)
        w_parts.append(jnp.reshape(wf, (_CT, _TOP_K)))
        i_parts.append(jnp.reshape(jf, (_CT, _TOP_K)))
    return (jnp.concatenate(w_parts, axis=0), jnp.concatenate(i_parts, axis=0))


# speedup vs baseline: 1.8339x; 1.8339x over previous
import functools

import jax
import jax.numpy as jnp
from jax import lax
from jax.experimental import pallas as pl
from jax.experimental.pallas import tpu as pltpu

_T = 8192
_D = 2048
_E = 64
_TOP_K = 8
_N_GROUPS = 8
_EPG = _E // _N_GROUPS
_TOPK_GROUPS = 4
_SCALE = 2.5

_BLK_T = 1024
_NEG = -1e30


def _router_block(x_ref, w_ref, b_ref, wts_ref, idx_ref):
    x = x_ref[...]
    w = w_ref[...]
    scores = jax.nn.sigmoid(jnp.dot(x, w, preferred_element_type=jnp.float32))
    st = jnp.transpose(scores)
    bt = st + b_ref[...]
    B = st.shape[1]

    st3 = jnp.reshape(st, (_N_GROUPS, _EPG, B))
    bt3 = jnp.reshape(bt, (_N_GROUPS, _EPG, B))
    riota = lax.broadcasted_iota(jnp.int32, (_EPG, B), 0)

    gsums = []
    for g in range(_N_GROUPS):
        sg = bt3[g]
        m1 = jnp.max(sg, axis=0, keepdims=True)
        i1 = jnp.min(jnp.where(sg == m1, riota, _EPG), axis=0, keepdims=True)
        m2 = jnp.max(jnp.where(riota == i1, _NEG, sg), axis=0, keepdims=True)
        gsums.append(m1 + m2)

    gs = jnp.concatenate(gsums, axis=0)
    giota = lax.broadcasted_iota(jnp.int32, (_N_GROUPS, B), 0)
    keep = jnp.zeros((_N_GROUPS, B), dtype=jnp.bool_)
    for _ in range(_TOPK_GROUPS):
        best = jnp.max(gs, axis=0, keepdims=True)
        cg = jnp.min(jnp.where(gs == best, giota, _N_GROUPS), axis=0,
                     keepdims=True)
        keep = keep | (giota == cg)
        gs = jnp.where(giota == cg, _NEG, gs)

    eidx = (lax.broadcasted_iota(jnp.int32, (_N_GROUPS, _EPG, B), 0) * _EPG
            + lax.broadcasted_iota(jnp.int32, (_N_GROUPS, _EPG, B), 1))
    keep3 = jnp.reshape(keep, (_N_GROUPS, 1, B))
    masked = jnp.where(keep3, bt3, 0.0)

    w_rows = []
    i_rows = []
    for _ in range(_TOP_K):
        m8 = jnp.max(masked, axis=1)
        m = jnp.max(m8, axis=0, keepdims=True)
        is_m = masked == jnp.reshape(m, (1, 1, B))
        ik8 = jnp.min(jnp.where(is_m, eidx, _E), axis=1)
        ik = jnp.min(ik8, axis=0, keepdims=True)
        sel = eidx == jnp.reshape(ik, (1, 1, B))
        wk8 = jnp.max(jnp.where(sel, st3, _NEG), axis=1)
        wk = jnp.max(wk8, axis=0, keepdims=True)
        masked = jnp.where(sel, _NEG, masked)
        w_rows.append(wk)
        i_rows.append(ik)

    wt = jnp.concatenate(w_rows, axis=0)
    it = jnp.concatenate(i_rows, axis=0)
    s = jnp.sum(wt, axis=0, keepdims=True)
    wt = wt * (_SCALE / (s + 1e-20))

    wts_ref[...] = jnp.transpose(wt)
    idx_ref[...] = jnp.transpose(it)


@jax.jit
def kernel(x, kernel_DE, bias_E):
    x = jnp.asarray(x, jnp.float32)
    bias_2d = jnp.reshape(bias_E, (_E, 1))
    grid = (_T // _BLK_T,)
    wts, idx = pl.pallas_call(
        _router_block,
        grid=grid,
        in_specs=[
            pl.BlockSpec((_BLK_T, _D), lambda i: (i, 0)),
            pl.BlockSpec((_D, _E), lambda i: (0, 0)),
            pl.BlockSpec((_E, 1), lambda i: (0, 0)),
        ],
        out_specs=[
            pl.BlockSpec((_BLK_T, _TOP_K), lambda i: (i, 0)),
            pl.BlockSpec((_BLK_T, _TOP_K), lambda i: (i, 0)),
        ],
        out_shape=[
            jax.ShapeDtypeStruct((_T, _TOP_K), jnp.float32),
            jax.ShapeDtypeStruct((_T, _TOP_K), jnp.int32),
        ],
    )(x, kernel_DE, bias_2d)
    return (wts, idx)
